# baseline (speedup 1.0000x reference)
import jax
import jax.numpy as jnp
import numpy as np
from jax.experimental import pallas as pl
from jax.experimental.pallas import tpu as pltpu

_B = 128
_V = 100000
_NS = 16
_SL = 6272
_ROWS = 8

_TINY = np.float32(np.finfo(np.float32).tiny)
_KS = (np.int32(0), np.int32(42), np.int32(np.uint32(0x1BD11BDA) ^ np.uint32(42)))
_ROT = ((13, 15, 26, 6), (17, 29, 16, 24))


def _sliced(x, red, comb):
    parts = [red(x[:, j * _SL:min((j + 1) * _SL, _V)], axis=-1,
                 keepdims=True) for j in range(_NS)]
    while len(parts) > 1:
        parts = [comb(parts[i], parts[i + 1])
                 for i in range(0, len(parts), 2)]
    return parts[0]


def _rsum(x):
    return _sliced(x, jnp.sum, jnp.add)


def _rmax(x):
    return _sliced(x, jnp.max, jnp.maximum)


def _rmin(x):
    return _sliced(x, jnp.min, jnp.minimum)


def _rotl(x, r):
    return jax.lax.shift_left(x, np.int32(r)) | jax.lax.shift_right_logical(
        x, np.int32(32 - r))


def _gumbel_bits(idx):
    x0 = jnp.zeros_like(idx) + _KS[0]
    x1 = idx + _KS[1]
    for i in range(5):
        for r in _ROT[i % 2]:
            x0 = x0 + x1
            x1 = _rotl(x1, r)
            x1 = x1 ^ x0
        x0 = x0 + _KS[(i + 1) % 3]
        x1 = x1 + _KS[(i + 2) % 3] + np.int32(i + 1)
    return x0 ^ x1


def _gumbel(idx):
    bits = _gumbel_bits(idx)
    fb = jax.lax.shift_right_logical(bits, np.int32(9)) | np.int32(0x3F800000)
    u = jax.lax.bitcast_convert_type(fb, jnp.float32) - 1.0
    u = jnp.maximum(_TINY, u * (np.float32(1.0) - _TINY) + _TINY)
    return -jnp.log(-jnp.log(u))


def _sampler_kernel(logits_ref, temp_ref, topk_ref, topp_ref,
                    minp_ref, out_ref):
    s = logits_ref[...] / temp_ref[...]
    m = _rmax(s)
    e = jnp.exp(s - m)
    kf = topk_ref[...]

    def body1(i, t):
        cand = t + jnp.left_shift(1, 29 - i)
        candf = jax.lax.bitcast_convert_type(cand, jnp.float32)
        cnt = _rsum(jnp.where(e >= candf, 1.0, 0.0))
        return jnp.where(cnt >= kf, cand, t)

    t1 = jax.lax.fori_loop(0, 30, body1, jnp.zeros((_ROWS, 1), jnp.int32))
    t1f = jax.lax.bitcast_convert_type(t1, jnp.float32)

    e1 = jnp.where(e >= t1f, e, 0.0)
    p_target = topp_ref[...] * _rsum(e1)

    def body2(i, t):
        cand = t + jnp.left_shift(1, 29 - i)
        candf = jax.lax.bitcast_convert_type(cand, jnp.float32)
        ssum = _rsum(jnp.where(e1 > candf, e1, 0.0))
        return jnp.where(ssum >= p_target, cand, t)

    tlf = jax.lax.fori_loop(0, 30, body2, jnp.zeros((_ROWS, 1), jnp.int32))
    t2f = jax.lax.bitcast_convert_type(tlf + 1, jnp.float32)
    thresh = jnp.maximum(t2f, minp_ref[...])

    iota = jax.lax.broadcasted_iota(jnp.int32, (_ROWS, _V), 1)
    rows = (pl.program_id(0) * _ROWS
            + jax.lax.broadcasted_iota(jnp.int32, (_ROWS, _V), 0))
    g = _gumbel(rows * _V + iota)
    z = jnp.where(e1 >= thresh, s + g, -jnp.inf)
    zmax = _rmax(z)
    out_ref[...] = _rmin(jnp.where(z == zmax, iota, _V))


@jax.jit
def kernel(logits, temperatures, top_ks, top_ps, min_ps):
    row = lambda x: x.reshape(_B, 1)
    grid = (_B // _ROWS,)
    wide = pl.BlockSpec((_ROWS, _V), lambda i: (i, 0))
    slim = pl.BlockSpec((_ROWS, 1), lambda i: (i, 0))
    tokens = pl.pallas_call(
        _sampler_kernel,
        grid=grid,
        in_specs=[wide, slim, slim, slim, slim],
        out_specs=slim,
        out_shape=jax.ShapeDtypeStruct((_B, 1), jnp.int32),
        compiler_params=pltpu.CompilerParams(
            dimension_semantics=("parallel",)),
    )(logits, row(temperatures), row(top_ks.astype(jnp.float32)),
      row(top_ps), row(min_ps))
    return tokens.reshape(_B)


# speedup vs baseline: 1.0551x; 1.0551x over previous
import jax
import jax.numpy as jnp
import numpy as np
from jax.experimental import pallas as pl
from jax.experimental.pallas import tpu as pltpu

_B = 128
_V = 100000
_VP = 100352
_CH = 1024
_NCH = _VP // _CH
_NS = 16
_SL = 6272
_ROWS = 8

_TINY = np.float32(np.finfo(np.float32).tiny)
_KS = (np.int32(0), np.int32(42),
       np.int32(np.uint32(0x1BD11BDA) ^ np.uint32(42)))
_ROT = ((13, 15, 26, 6), (17, 29, 16, 24))


def _sliced(x, red, comb):
    n = x.shape[-1]
    parts = [red(x[:, j * _SL:min((j + 1) * _SL, n)], axis=-1,
                 keepdims=True) for j in range(_NS)]
    while len(parts) > 1:
        parts = [comb(parts[i], parts[i + 1])
                 for i in range(0, len(parts), 2)]
    return parts[0]


def _rsum(x):
    return _sliced(x, jnp.sum, jnp.add)


def _rmax(x):
    return _sliced(x, jnp.max, jnp.maximum)


def _rotl(x, r):
    return jax.lax.shift_left(x, np.int32(r)) | jax.lax.shift_right_logical(
        x, np.int32(32 - r))


def _gumbel(idx):
    x0 = jnp.zeros_like(idx) + _KS[0]
    x1 = idx + _KS[1]
    for i in range(5):
        for r in _ROT[i % 2]:
            x0 = x0 + x1
            x1 = _rotl(x1, r)
            x1 = x1 ^ x0
        x0 = x0 + _KS[(i + 1) % 3]
        x1 = x1 + _KS[(i + 2) % 3] + np.int32(i + 1)
    bits = x0 ^ x1
    fb = jax.lax.shift_right_logical(bits, np.int32(9)) | np.int32(0x3F800000)
    u = jax.lax.bitcast_convert_type(fb, jnp.float32) - 1.0
    u = jnp.maximum(_TINY, u * (np.float32(1.0) - _TINY) + _TINY)
    return -jnp.log(-jnp.log(u))


def _sampler_kernel(logits_ref, temp_ref, topk_ref, topp_ref, minp_ref,
                    out_ref, s_ref, e1_ref):
    s = logits_ref[...] / temp_ref[...]
    m = _rmax(s)
    e = jnp.exp(s - m)
    kf = topk_ref[...]

    def body1(i, t):
        cand = t + jnp.left_shift(1, 29 - i)
        candf = jax.lax.bitcast_convert_type(cand, jnp.float32)
        cnt = _rsum(jnp.where(e >= candf, 1.0, 0.0))
        return jnp.where(cnt >= kf, cand, t)

    t1 = jax.lax.fori_loop(0, 30, body1, jnp.zeros((_ROWS, 1), jnp.int32))
    t1f = jax.lax.bitcast_convert_type(t1, jnp.float32)

    e1 = jnp.where(e >= t1f, e, 0.0)
    p_target = topp_ref[...] * _rsum(e1)

    def body2(i, t):
        cand = t + jnp.left_shift(1, 29 - i)
        candf = jax.lax.bitcast_convert_type(cand, jnp.float32)
        ssum = _rsum(jnp.where(e1 > candf, e1, 0.0))
        return jnp.where(ssum >= p_target, cand, t)

    tlf = jax.lax.fori_loop(0, 30, body2, jnp.zeros((_ROWS, 1), jnp.int32))
    t2f = jax.lax.bitcast_convert_type(tlf + 1, jnp.float32)
    thresh = jnp.maximum(t2f, minp_ref[...])

    s_ref[:, :_V] = s
    s_ref[:, _V:] = jnp.zeros((_ROWS, _VP - _V), jnp.float32)
    e1_ref[:, :_V] = e1
    e1_ref[:, _V:] = jnp.zeros((_ROWS, _VP - _V), jnp.float32)

    rowbase = ((pl.program_id(0) * _ROWS
                + jax.lax.broadcasted_iota(jnp.int32, (_ROWS, _CH), 0))
               * _V)
    lane = jax.lax.broadcasted_iota(jnp.int32, (_ROWS, _CH), 1)

    def chunk_body(c, carry):
        zm, zi = carry
        off = c * _CH
        e1c = e1_ref[:, pl.ds(off, _CH)]
        sc = s_ref[:, pl.ds(off, _CH)]
        col = off + lane
        g = _gumbel(rowbase + col)
        z = jnp.where(e1c >= thresh, sc + g, -jnp.inf)
        upd = z > zm
        return jnp.where(upd, z, zm), jnp.where(upd, col, zi)

    zm0 = jnp.full((_ROWS, _CH), -jnp.inf, jnp.float32)
    zi0 = jnp.full((_ROWS, _CH), _V, jnp.int32)
    zm, zi = jax.lax.fori_loop(0, _NCH, chunk_body, (zm0, zi0))
    zbest = jnp.max(zm, axis=-1, keepdims=True)
    out_ref[...] = jnp.min(jnp.where(zm == zbest, zi, _V), axis=-1,
                           keepdims=True)


@jax.jit
def kernel(logits, temperatures, top_ks, top_ps, min_ps):
    row = lambda x: x.reshape(_B, 1)
    grid = (_B // _ROWS,)
    wide = pl.BlockSpec((_ROWS, _V), lambda i: (i, 0))
    slim = pl.BlockSpec((_ROWS, 1), lambda i: (i, 0))
    tokens = pl.pallas_call(
        _sampler_kernel,
        grid=grid,
        in_specs=[wide, slim, slim, slim, slim],
        out_specs=slim,
        out_shape=jax.ShapeDtypeStruct((_B, 1), jnp.int32),
        scratch_shapes=[pltpu.VMEM((_ROWS, _VP), jnp.float32),
                        pltpu.VMEM((_ROWS, _VP), jnp.float32)],
        compiler_params=pltpu.CompilerParams(
            dimension_semantics=("parallel",)),
    )(logits, row(temperatures), row(top_ks.astype(jnp.float32)),
      row(top_ps), row(min_ps))
    return tokens.reshape(_B)


# speedup vs baseline: 1.1931x; 1.1309x over previous
import jax
import jax.numpy as jnp
import numpy as np
from jax.experimental import pallas as pl
from jax.experimental.pallas import tpu as pltpu

_B = 128
_V = 100000
_VP = 100352
_CH = 1024
_NCH = _VP // _CH
_NS = 16
_SL = 6272
_ROWS = 16

_TINY = np.float32(np.finfo(np.float32).tiny)
_KS = (np.int32(0), np.int32(42),
       np.int32(np.uint32(0x1BD11BDA) ^ np.uint32(42)))
_ROT = ((13, 15, 26, 6), (17, 29, 16, 24))


def _sliced(x, red, comb):
    n = x.shape[-1]
    parts = [red(x[:, j * _SL:min((j + 1) * _SL, n)], axis=-1,
                 keepdims=True) for j in range(_NS)]
    while len(parts) > 1:
        parts = [comb(parts[i], parts[i + 1])
                 for i in range(0, len(parts), 2)]
    return parts[0]


def _rsum(x):
    return _sliced(x, jnp.sum, jnp.add)


def _rmax(x):
    return _sliced(x, jnp.max, jnp.maximum)


def _rotl(x, r):
    return jax.lax.shift_left(x, np.int32(r)) | jax.lax.shift_right_logical(
        x, np.int32(32 - r))


def _gumbel(idx):
    x0 = jnp.zeros_like(idx) + _KS[0]
    x1 = idx + _KS[1]
    for i in range(5):
        for r in _ROT[i % 2]:
            x0 = x0 + x1
            x1 = _rotl(x1, r)
            x1 = x1 ^ x0
        x0 = x0 + _KS[(i + 1) % 3]
        x1 = x1 + _KS[(i + 2) % 3] + np.int32(i + 1)
    bits = x0 ^ x1
    fb = jax.lax.shift_right_logical(bits, np.int32(9)) | np.int32(0x3F800000)
    u = jax.lax.bitcast_convert_type(fb, jnp.float32) - 1.0
    u = jnp.maximum(_TINY, u * (np.float32(1.0) - _TINY) + _TINY)
    return -jnp.log(-jnp.log(u))


def _sampler_kernel(logits_ref, temp_ref, topk_ref, topp_ref, minp_ref,
                    out_ref, s_ref, e1_ref):
    s = logits_ref[...] / temp_ref[...]
    m = _rmax(s)
    e = jnp.exp(s - m)
    kf = topk_ref[...]

    def body1(i, t):
        cand = t + jnp.left_shift(1, 29 - i)
        candf = jax.lax.bitcast_convert_type(cand, jnp.float32)
        cnt = _rsum(jnp.where(e >= candf, 1.0, 0.0))
        return jnp.where(cnt >= kf, cand, t)

    t1 = jax.lax.fori_loop(0, 30, body1, jnp.zeros((_ROWS, 1), jnp.int32))
    t1f = jax.lax.bitcast_convert_type(t1, jnp.float32)

    e1 = jnp.where(e >= t1f, e, 0.0)
    p_target = topp_ref[...] * _rsum(e1)

    def body2(i, t):
        cand = t + jnp.left_shift(1, 29 - i)
        candf = jax.lax.bitcast_convert_type(cand, jnp.float32)
        ssum = _rsum(jnp.where(e1 > candf, e1, 0.0))
        return jnp.where(ssum >= p_target, cand, t)

    tlf = jax.lax.fori_loop(0, 30, body2, jnp.zeros((_ROWS, 1), jnp.int32))
    t2f = jax.lax.bitcast_convert_type(tlf + 1, jnp.float32)
    thresh = jnp.maximum(t2f, minp_ref[...])

    s_ref[:, :_V] = s
    s_ref[:, _V:] = jnp.zeros((_ROWS, _VP - _V), jnp.float32)
    e1_ref[:, :_V] = e1
    e1_ref[:, _V:] = jnp.zeros((_ROWS, _VP - _V), jnp.float32)

    rowbase = ((pl.program_id(0) * _ROWS
                + jax.lax.broadcasted_iota(jnp.int32, (_ROWS, _CH), 0))
               * _V)
    lane = jax.lax.broadcasted_iota(jnp.int32, (_ROWS, _CH), 1)

    def chunk_body(c, carry):
        zm, zi = carry
        off = c * _CH
        e1c = e1_ref[:, pl.ds(off, _CH)]
        sc = s_ref[:, pl.ds(off, _CH)]
        col = off + lane
        g = _gumbel(rowbase + col)
        z = jnp.where(e1c >= thresh, sc + g, -jnp.inf)
        upd = z > zm
        return jnp.where(upd, z, zm), jnp.where(upd, col, zi)

    zm0 = jnp.full((_ROWS, _CH), -jnp.inf, jnp.float32)
    zi0 = jnp.full((_ROWS, _CH), _V, jnp.int32)
    zm, zi = jax.lax.fori_loop(0, _NCH, chunk_body, (zm0, zi0))
    zbest = jnp.max(zm, axis=-1, keepdims=True)
    out_ref[...] = jnp.min(jnp.where(zm == zbest, zi, _V), axis=-1,
                           keepdims=True)


@jax.jit
def kernel(logits, temperatures, top_ks, top_ps, min_ps):
    row = lambda x: x.reshape(_B, 1)
    grid = (_B // _ROWS,)
    wide = pl.BlockSpec((_ROWS, _V), lambda i: (i, 0))
    slim = pl.BlockSpec((_ROWS, 1), lambda i: (i, 0))
    tokens = pl.pallas_call(
        _sampler_kernel,
        grid=grid,
        in_specs=[wide, slim, slim, slim, slim],
        out_specs=slim,
        out_shape=jax.ShapeDtypeStruct((_B, 1), jnp.int32),
        scratch_shapes=[pltpu.VMEM((_ROWS, _VP), jnp.float32),
                        pltpu.VMEM((_ROWS, _VP), jnp.float32)],
        compiler_params=pltpu.CompilerParams(
            dimension_semantics=("parallel",)),
    )(logits, row(temperatures), row(top_ks.astype(jnp.float32)),
      row(top_ps), row(min_ps))
    return tokens.reshape(_B)


# speedup vs baseline: 1.2124x; 1.0162x over previous
import jax
import jax.numpy as jnp
import numpy as np
from jax.experimental import pallas as pl
from jax.experimental.pallas import tpu as pltpu

_B = 128
_V = 100000
_CH = 1024
_NFULL = _V // _CH
_NS = 16
_SL = 6272
_ROWS = 32

_TINY = np.float32(np.finfo(np.float32).tiny)
_KS = (np.int32(0), np.int32(42),
       np.int32(np.uint32(0x1BD11BDA) ^ np.uint32(42)))
_ROT = ((13, 15, 26, 6), (17, 29, 16, 24))


def _sliced(x, red, comb):
    n = x.shape[-1]
    parts = [red(x[:, j * _SL:min((j + 1) * _SL, n)], axis=-1,
                 keepdims=True) for j in range(_NS)]
    while len(parts) > 1:
        parts = [comb(parts[i], parts[i + 1])
                 for i in range(0, len(parts), 2)]
    return parts[0]


def _rsum(x):
    return _sliced(x, jnp.sum, jnp.add)


def _rmax(x):
    return _sliced(x, jnp.max, jnp.maximum)


def _rotl(x, r):
    return jax.lax.shift_left(x, np.int32(r)) | jax.lax.shift_right_logical(
        x, np.int32(32 - r))


def _gumbel(idx):
    x0 = jnp.zeros_like(idx) + _KS[0]
    x1 = idx + _KS[1]
    for i in range(5):
        for r in _ROT[i % 2]:
            x0 = x0 + x1
            x1 = _rotl(x1, r)
            x1 = x1 ^ x0
        x0 = x0 + _KS[(i + 1) % 3]
        x1 = x1 + _KS[(i + 2) % 3] + np.int32(i + 1)
    bits = x0 ^ x1
    fb = jax.lax.shift_right_logical(bits, np.int32(9)) | np.int32(0x3F800000)
    u = jax.lax.bitcast_convert_type(fb, jnp.float32) - 1.0
    u = jnp.maximum(_TINY, u * (np.float32(1.0) - _TINY) + _TINY)
    return -jnp.log(-jnp.log(u))


def _sampler_kernel(logits_ref, temp_ref, topk_ref, topp_ref, minp_ref,
                    out_ref):
    s = logits_ref[...] / temp_ref[...]
    m = _rmax(s)
    e = jnp.exp(s - m)
    kf = topk_ref[...]

    def body1(i, t):
        cand = t + jnp.left_shift(1, 29 - i)
        candf = jax.lax.bitcast_convert_type(cand, jnp.float32)
        cnt = _rsum(jnp.where(e >= candf, 1.0, 0.0))
        return jnp.where(cnt >= kf, cand, t)

    t1 = jax.lax.fori_loop(0, 30, body1, jnp.zeros((_ROWS, 1), jnp.int32))
    t1f = jax.lax.bitcast_convert_type(t1, jnp.float32)

    e1 = jnp.where(e >= t1f, e, 0.0)
    p_target = topp_ref[...] * _rsum(e1)

    def body2(i, t):
        cand = t + jnp.left_shift(1, 29 - i)
        candf = jax.lax.bitcast_convert_type(cand, jnp.float32)
        ssum = _rsum(jnp.where(e1 > candf, e1, 0.0))
        return jnp.where(ssum >= p_target, cand, t)

    tlf = jax.lax.fori_loop(0, 30, body2, jnp.zeros((_ROWS, 1), jnp.int32))
    t2f = jax.lax.bitcast_convert_type(tlf + 1, jnp.float32)
    thresh = jnp.maximum(t2f, minp_ref[...])

    rowoff = pl.program_id(0) * _ROWS
    invt = temp_ref[...]

    def scan_chunk(off, width, carry):
        zm, zi = carry
        lc = logits_ref[:, pl.ds(off, width)]
        sc = lc / invt
        ec = jnp.exp(sc - m)
        col = off + jax.lax.broadcasted_iota(jnp.int32, (_ROWS, width), 1)
        rb = (rowoff
              + jax.lax.broadcasted_iota(jnp.int32, (_ROWS, width), 0)) * _V
        g = _gumbel(rb + col)
        z = jnp.where(ec >= thresh, sc + g, -jnp.inf)
        upd = z > zm
        return jnp.where(upd, z, zm), jnp.where(upd, col, zi)

    zm0 = jnp.full((_ROWS, _CH), -jnp.inf, jnp.float32)
    zi0 = jnp.full((_ROWS, _CH), _V, jnp.int32)
    zm, zi = jax.lax.fori_loop(
        0, _NFULL, lambda c, car: scan_chunk(c * _CH, _CH, car),
        (zm0, zi0))
    tw = _V - _NFULL * _CH
    tm, ti = scan_chunk(_NFULL * _CH, tw,
                        (jnp.full((_ROWS, tw), -jnp.inf, jnp.float32),
                         jnp.full((_ROWS, tw), _V, jnp.int32)))
    zbest = jnp.maximum(jnp.max(zm, axis=-1, keepdims=True),
                        jnp.max(tm, axis=-1, keepdims=True))
    out_ref[...] = jnp.minimum(
        jnp.min(jnp.where(zm == zbest, zi, _V), axis=-1, keepdims=True),
        jnp.min(jnp.where(tm == zbest, ti, _V), axis=-1, keepdims=True))


@jax.jit
def kernel(logits, temperatures, top_ks, top_ps, min_ps):
    row = lambda x: x.reshape(_B, 1)
    grid = (_B // _ROWS,)
    wide = pl.BlockSpec((_ROWS, _V), lambda i: (i, 0))
    slim = pl.BlockSpec((_ROWS, 1), lambda i: (i, 0))
    tokens = pl.pallas_call(
        _sampler_kernel,
        grid=grid,
        in_specs=[wide, slim, slim, slim, slim],
        out_specs=slim,
        out_shape=jax.ShapeDtypeStruct((_B, 1), jnp.int32),
        compiler_params=pltpu.CompilerParams(
            dimension_semantics=("parallel",)),
    )(logits, row(temperatures), row(top_ks.astype(jnp.float32)),
      row(top_ps), row(min_ps))
    return tokens.reshape(_B)


# speedup vs baseline: 1.2126x; 1.0001x over previous
import jax
import jax.numpy as jnp
import numpy as np
from jax.experimental import pallas as pl
from jax.experimental.pallas import tpu as pltpu

_B = 128
_V = 100000
_CH = 1024
_NFULL = _V // _CH
_NS = 16
_SL = 6272
_ROWS = 32

_TINY = np.float32(np.finfo(np.float32).tiny)
_KS = (np.int32(0), np.int32(42),
       np.int32(np.uint32(0x1BD11BDA) ^ np.uint32(42)))
_ROT = ((13, 15, 26, 6), (17, 29, 16, 24))


def _sliced(x, red, comb):
    n = x.shape[-1]
    parts = [red(x[:, j * _SL:min((j + 1) * _SL, n)], axis=-1,
                 keepdims=True) for j in range(_NS)]
    while len(parts) > 1:
        parts = [comb(parts[i], parts[i + 1])
                 for i in range(0, len(parts), 2)]
    return parts[0]


def _rsum(x):
    return _sliced(x, jnp.sum, jnp.add)


def _rmax(x):
    return _sliced(x, jnp.max, jnp.maximum)


def _rotl(x, r):
    return jax.lax.shift_left(x, np.int32(r)) | jax.lax.shift_right_logical(
        x, np.int32(32 - r))


def _gumbel(idx):
    x0 = jnp.zeros_like(idx) + _KS[0]
    x1 = idx + _KS[1]
    for i in range(5):
        for r in _ROT[i % 2]:
            x0 = x0 + x1
            x1 = _rotl(x1, r)
            x1 = x1 ^ x0
        x0 = x0 + _KS[(i + 1) % 3]
        x1 = x1 + _KS[(i + 2) % 3] + np.int32(i + 1)
    bits = x0 ^ x1
    fb = jax.lax.shift_right_logical(bits, np.int32(9)) | np.int32(0x3F800000)
    u = jax.lax.bitcast_convert_type(fb, jnp.float32) - 1.0
    u = jnp.maximum(_TINY, u * (np.float32(1.0) - _TINY) + _TINY)
    return -jnp.log(-jnp.log(u))


def _sampler_kernel(logits_ref, temp_ref, topk_ref, topp_ref, minp_ref,
                    out_ref):
    s = logits_ref[...] / temp_ref[...]
    m = _rmax(s)
    e = jnp.exp(s - m)
    kf = topk_ref[...]

    def body1(i, t):
        cand = t + jnp.left_shift(1, 29 - i)
        candf = jax.lax.bitcast_convert_type(cand, jnp.float32)
        cnt = _rsum(jnp.where(e >= candf, 1.0, 0.0))
        return jnp.where(cnt >= kf, cand, t)

    t1 = jax.lax.fori_loop(0, 30, body1, jnp.zeros((_ROWS, 1), jnp.int32))
    t1f = jax.lax.bitcast_convert_type(t1, jnp.float32)

    e1 = jnp.where(e >= t1f, e, 0.0)
    p_target = topp_ref[...] * _rsum(e1)

    def body2(i, t):
        cand = t + jnp.left_shift(1, 29 - i)
        candf = jax.lax.bitcast_convert_type(cand, jnp.float32)
        ssum = _rsum(jnp.where(e1 > candf, e1, 0.0))
        return jnp.where(ssum >= p_target, cand, t)

    tlf = jax.lax.fori_loop(0, 30, body2, jnp.zeros((_ROWS, 1), jnp.int32))
    t2f = jax.lax.bitcast_convert_type(tlf + 1, jnp.float32)
    thresh = jnp.maximum(t2f, minp_ref[...])

    rowoff = pl.program_id(0) * _ROWS
    tempv = temp_ref[...]

    def scan_chunk(off, width, carry):
        zm, zi = carry
        lc = logits_ref[:, pl.ds(off, width)]
        sc = lc / tempv
        ec = jnp.exp(sc - m)
        col = off + jax.lax.broadcasted_iota(jnp.int32, (_ROWS, width), 1)
        rb = (rowoff
              + jax.lax.broadcasted_iota(jnp.int32, (_ROWS, width), 0)) * _V
        g = _gumbel(rb + col)
        z = jnp.where(ec >= thresh, sc + g, -jnp.inf)
        upd = z > zm
        return jnp.where(upd, z, zm), jnp.where(upd, col, zi)

    zm0 = jnp.full((_ROWS, _CH), -jnp.inf, jnp.float32)
    zi0 = jnp.full((_ROWS, _CH), _V, jnp.int32)
    zm, zi = jax.lax.fori_loop(
        0, _NFULL, lambda c, car: scan_chunk(c * _CH, _CH, car),
        (zm0, zi0))
    tw = _V - _NFULL * _CH
    tm, ti = scan_chunk(_NFULL * _CH, tw,
                        (jnp.full((_ROWS, tw), -jnp.inf, jnp.float32),
                         jnp.full((_ROWS, tw), _V, jnp.int32)))
    zbest = jnp.maximum(jnp.max(zm, axis=-1, keepdims=True),
                        jnp.max(tm, axis=-1, keepdims=True))
    out_ref[...] = jnp.minimum(
        jnp.min(jnp.where(zm == zbest, zi, _V), axis=-1, keepdims=True),
        jnp.min(jnp.where(tm == zbest, ti, _V), axis=-1, keepdims=True))


@jax.jit
def kernel(logits, temperatures, top_ks, top_ps, min_ps):
    row = lambda x: x.reshape(_B, 1)
    grid = (_B // _ROWS,)
    wide = pl.BlockSpec((_ROWS, _V), lambda i: (i, 0))
    slim = pl.BlockSpec((_ROWS, 1), lambda i: (i, 0))
    tokens = pl.pallas_call(
        _sampler_kernel,
        grid=grid,
        in_specs=[wide, slim, slim, slim, slim],
        out_specs=slim,
        out_shape=jax.ShapeDtypeStruct((_B, 1), jnp.int32),
        compiler_params=pltpu.CompilerParams(
            dimension_semantics=("parallel",)),
    )(logits, row(temperatures), row(top_ks.astype(jnp.float32)),
      row(top_ps), row(min_ps))
    return tokens.reshape(_B)
